# Initial kernel scaffold; baseline (speedup 1.0000x reference)
import jax
import jax.numpy as jnp
from jax.experimental import pallas as pl


def kernel(edges, relative_poses, nodes):
    raise NotImplementedError("write your pallas kernel here")


# speedup vs baseline: 7.8101x; 7.8101x over previous
import functools

import jax
import jax.numpy as jnp
from jax import lax
from jax.experimental import pallas as pl
from jax.experimental.pallas import tpu as pltpu
from jax.experimental.pallas import tpu_sc as plsc

_EPS = 1e-6
_PI = 3.14159265358979323846
_HALF_PI = _PI / 2.0
_ATAN_C = (
    0.9999966347006731,
    -0.3331830289944677,
    0.19813213509068275,
    -0.1324752277162814,
    0.07981120495618609,
    -0.03372593810415406,
    0.006842624897572022,
)

_C = 1600
_IW = 100
_IROWS = _C // _IW


def _rsqrt(x):
    i = lax.bitcast_convert_type(x, jnp.int32)
    i = jnp.int32(0x5F3759DF) - lax.shift_right_arithmetic(i, 1)
    y = lax.bitcast_convert_type(i, jnp.float32)
    xh = x * 0.5
    y = y * (1.5 - xh * y * y)
    y = y * (1.5 - xh * y * y)
    y = y * (1.5 - xh * y * y)
    return y


def _atan2_pos(n, w):
    aw = jnp.abs(w)
    mn = jnp.minimum(n, aw)
    mx = jnp.maximum(n, aw)
    z = mn / jnp.maximum(mx, 1e-35)
    z2 = z * z
    p = jnp.float32(_ATAN_C[-1])
    for c in _ATAN_C[-2::-1]:
        p = p * z2 + jnp.float32(c)
    p = p * z
    r = jnp.where(n > aw, _HALF_PI - p, p)
    return jnp.where(w >= 0.0, r, _PI - r)


def _se3_log_parts(tx, ty, tz, qx, qy, qz, qw):
    n2 = qx * qx + qy * qy + qz * qz
    w2 = qw * qw
    s2 = n2 + w2
    n = n2 * _rsqrt(jnp.maximum(n2, 1e-35))
    theta = 2.0 * _atan2_pos(n, qw)
    n_safe = jnp.where(n > _EPS, n, 1.0)
    w_safe = jnp.where(jnp.abs(qw) > _EPS, qw, 1.0)
    scale = jnp.where(n > _EPS, theta / n_safe, 2.0 / w_safe)
    px = scale * qx
    py = scale * qy
    pz = scale * qz
    th = jnp.abs(scale) * n
    th_safe = jnp.where(th > _EPS, th, 1.0)
    inv_s2 = 1.0 / s2
    cth = (w2 - n2) * inv_s2
    sth = 2.0 * n * qw * inv_s2
    coef = jnp.where(
        th > _EPS,
        1.0 / (th_safe * th_safe)
        - (1.0 + cth) / (2.0 * th_safe * sth),
        1.0 / 12.0,
    )
    pvx = py * tz - pz * ty
    pvy = pz * tx - px * tz
    pvz = px * ty - py * tx
    ppvx = py * pvz - pz * pvy
    ppvy = pz * pvx - px * pvz
    ppvz = px * pvy - py * pvx
    rx = tx - 0.5 * pvx + coef * ppvx
    ry = ty - 0.5 * pvy + coef * ppvy
    rz = tz - 0.5 * pvz + coef * ppvz
    return rx, ry, rz, px, py, pz


def _edge_error_log(rel, n1, n2c):
    rtx, rty, rtz, rqx, rqy, rqz, rqw = rel
    t1x, t1y, t1z, ax, ay, az, aw = n1
    t2x, t2y, t2z, bx, by, bz, bw = n2c
    qbx = -rqw * ax + rqx * aw - rqy * az + rqz * ay
    qby = -rqw * ay + rqx * az + rqy * aw - rqz * ax
    qbz = -rqw * az - rqx * ay + rqy * ax + rqz * aw
    qbw = rqw * aw + rqx * ax + rqy * ay + rqz * az
    vx = t2x - t1x
    vy = t2y - t1y
    vz = t2z - t1z
    uvx = qby * vz - qbz * vy
    uvy = qbz * vx - qbx * vz
    uvz = qbx * vy - qby * vx
    tex = rtx + vx + 2.0 * (qbw * uvx + qby * uvz - qbz * uvy)
    tey = rty + vy + 2.0 * (qbw * uvy + qbz * uvx - qbx * uvz)
    tez = rtz + vz + 2.0 * (qbw * uvz + qbx * uvy - qby * uvx)
    qex = qbw * bx + qbx * bw + qby * bz - qbz * by
    qey = qbw * by - qbx * bz + qby * bw + qbz * bx
    qez = qbw * bz + qbx * by - qby * bx + qbz * bw
    qew = qbw * bw - qbx * bx - qby * by - qbz * bz
    return _se3_log_parts(tex, tey, tez, qex, qey, qez, qew)


def _sc_body(idx1_hbm, idx2_hbm, rel_hbm, nodes_hbm, prior_out, err_out,
             idx1_v, idx2_v, rel_v, n1_v, n2_v, out_v, p_row, p_out, sem):
    nsc = lax.axis_index("c")
    wid = lax.axis_index("s") * 2 + nsc
    e_total = err_out.shape[0]
    per_w = e_total // 32
    n_chunks = per_w // _C
    lanes = lax.iota(jnp.int32, 16)
    zeros16 = jnp.zeros((16,), jnp.int32)

    def chunk_body(k, _):
        off = pl.multiple_of(wid * per_w + k * _C, _C)
        irow0 = pl.multiple_of(off // _IW, _IROWS)
        pltpu.sync_copy(idx1_hbm.at[pl.ds(irow0, _IROWS)], idx1_v)
        pltpu.sync_copy(idx2_hbm.at[pl.ds(irow0, _IROWS)], idx2_v)
        pltpu.sync_copy(rel_hbm.at[pl.ds(off, _C)], rel_v)
        descs = []
        for j in range(_IROWS):
            descs.append(pltpu.async_copy(
                nodes_hbm.at[idx1_v.at[j]],
                n1_v.at[pl.ds(j * _IW, _IW)], sem))
            descs.append(pltpu.async_copy(
                nodes_hbm.at[idx2_v.at[j]],
                n2_v.at[pl.ds(j * _IW, _IW)], sem))
        for d in descs:
            d.wait()

        def group_body(i, _):
            row = i * 16 + lanes

            def comp(ref, c):
                return plsc.load_gather(ref, [row, jnp.full((16,), c, jnp.int32)])

            rel = tuple(comp(rel_v, c) for c in range(7))
            g1 = tuple(comp(n1_v, c) for c in range(7))
            g2 = tuple(comp(n2_v, c) for c in range(7))
            res = _edge_error_log(rel, g1, g2)
            for c in range(6):
                plsc.store_scatter(out_v, [row, jnp.full((16,), c, jnp.int32)], res[c])
            return ()

        lax.fori_loop(0, _C // 16, group_body, (), unroll=False)
        pltpu.sync_copy(out_v, err_out.at[pl.ds(off, _C)])
        return ()

    lax.fori_loop(0, n_chunks, chunk_body, (), unroll=False)

    @pl.when(wid == 0)
    def _():
        pltpu.sync_copy(nodes_hbm.at[pl.ds(0, 1)], p_row)
        comps = tuple(
            plsc.load_gather(p_row, [zeros16, jnp.full((16,), c, jnp.int32)])
            for c in range(7))
        res = _se3_log_parts(*comps)
        mask0 = lanes == 0
        for c in range(6):
            plsc.store_scatter(p_out, [jnp.full((16,), c, jnp.int32)],
                               res[c], mask=mask0)
        pltpu.sync_copy(p_out, prior_out)


def kernel(edges, relative_poses, nodes):
    e_total = edges.shape[0]
    n_nodes = nodes.shape[0]
    nodes_pad = jnp.concatenate(
        [nodes, jnp.zeros((n_nodes, 1), jnp.float32)], axis=1)
    idx1 = edges[:, 0].reshape(-1, _IW)
    idx2 = edges[:, 1].reshape(-1, _IW)

    mesh = plsc.VectorSubcoreMesh(core_axis_name="c", subcore_axis_name="s")
    sc = pl.kernel(
        _sc_body,
        out_type=(
            jax.ShapeDtypeStruct((16,), jnp.float32),
            jax.ShapeDtypeStruct((e_total, 6), jnp.float32),
        ),
        mesh=mesh,
        compiler_params=pltpu.CompilerParams(
            needs_layout_passes=False, use_tc_tiling_on_sc=False),
        scratch_types=[
            pltpu.VMEM((_IROWS, _IW), jnp.int32),
            pltpu.VMEM((_IROWS, _IW), jnp.int32),
            pltpu.VMEM((_C, 7), jnp.float32),
            pltpu.VMEM((_C, 8), jnp.float32),
            pltpu.VMEM((_C, 8), jnp.float32),
            pltpu.VMEM((_C, 6), jnp.float32),
            pltpu.VMEM((1, 8), jnp.float32),
            pltpu.VMEM((16,), jnp.float32),
            pltpu.SemaphoreType.DMA,
        ],
    )
    prior16, err = sc(idx1, idx2, relative_poses, nodes_pad)
    return prior16[:6], err
